# baseline (device time: 44288 ns/iter reference)
import jax
import jax.numpy as jnp
from jax import lax
from jax.experimental import pallas as pl
from jax.experimental.pallas import tpu as pltpu

N_DEV = 4
S = 2
Q = 4


def kernel(x):
    _, m, n = x.shape
    xs = x.reshape(m, n)
    chunk = n // N_DEV
    half = chunk // 2
    rows = m // S
    qrows = m // Q

    def body(x_hbm, out_hbm, x_vmem, copy_sem, comm_r, comm_l,
             send_r, recv_r, send_l, recv_l,
             send_r2, recv_r2, send_l2, recv_l2,
             out_vmem, out_sems):
        p = lax.axis_index("i")
        left = lax.rem(p + N_DEV - 1, N_DEV)
        right = lax.rem(p + 1, N_DEV)

        staging = pltpu.make_async_copy(x_hbm, x_vmem, copy_sem)
        staging.start()

        barrier_sem = pltpu.get_barrier_semaphore()
        for nbr in [left, right]:
            pl.semaphore_signal(
                barrier_sem, inc=1,
                device_id=(nbr,), device_id_type=pl.DeviceIdType.MESH,
            )
        pl.semaphore_wait(barrier_sem, 2)

        def col_r(h):
            return lax.rem(p + 2 * N_DEV - 2 - h, N_DEV) * chunk

        def col_l(h):
            return lax.rem(p + 2 + h, N_DEV) * chunk + half

        def make(h, s, direction):
            if direction == 0:
                if h == 0:
                    src = x_hbm.at[pl.ds(s * rows, rows),
                                   pl.ds(lax.rem(p + N_DEV - 1, N_DEV) * chunk,
                                         half)]
                else:
                    src = comm_r.at[h, s]
                return pltpu.make_async_remote_copy(
                    src_ref=src,
                    dst_ref=comm_r.at[h + 1, s],
                    send_sem=send_r.at[h, s],
                    recv_sem=recv_r.at[h, s],
                    device_id=(right,),
                    device_id_type=pl.DeviceIdType.MESH,
                )
            else:
                if h == 0:
                    src = x_hbm.at[pl.ds(s * rows, rows),
                                   pl.ds(lax.rem(p + 1, N_DEV) * chunk + half,
                                         half)]
                else:
                    src = comm_l.at[h, s]
                return pltpu.make_async_remote_copy(
                    src_ref=src,
                    dst_ref=comm_l.at[h + 1, s],
                    send_sem=send_l.at[h, s],
                    recv_sem=recv_l.at[h, s],
                    device_id=(left,),
                    device_id_type=pl.DeviceIdType.MESH,
                )

        def make_last(q, direction):
            s = q // (Q // S)
            qq = q % (Q // S)
            rq = pl.ds(qq * qrows, qrows)
            if direction == 0:
                return pltpu.make_async_remote_copy(
                    src_ref=comm_r.at[2, s, rq],
                    dst_ref=comm_r.at[3, s, rq],
                    send_sem=send_r2.at[q],
                    recv_sem=recv_r2.at[q],
                    device_id=(right,),
                    device_id_type=pl.DeviceIdType.MESH,
                )
            else:
                return pltpu.make_async_remote_copy(
                    src_ref=comm_l.at[2, s, rq],
                    dst_ref=comm_l.at[3, s, rq],
                    send_sem=send_l2.at[q],
                    recv_sem=recv_l2.at[q],
                    device_id=(left,),
                    device_id_type=pl.DeviceIdType.MESH,
                )

        rdmas = {}
        out_copies = []
        for s in range(S):
            for d in (0, 1):
                rdmas[(0, s, d)] = make(0, s, d)
                rdmas[(0, s, d)].start()

        staging.wait()

        for h in range(2):
            for s in range(S):
                rsl = pl.ds(s * rows, rows)
                for d in (0, 1):
                    comm = comm_r if d == 0 else comm_l
                    col = col_r(h) if d == 0 else col_l(h)
                    rdmas[(h, s, d)].wait_recv()
                    comm[h + 1, s] = comm[h + 1, s] + x_vmem[rsl, pl.ds(col, half)]
                    if h == 0:
                        rdmas[(1, s, d)] = make(1, s, d)
                        rdmas[(1, s, d)].start()
                    else:
                        for q in range(s * (Q // S), (s + 1) * (Q // S)):
                            rdmas[(2, q, d)] = make_last(q, d)
                            rdmas[(2, q, d)].start()

        for q in range(Q):
            s = q // (Q // S)
            rq = pl.ds((q % (Q // S)) * qrows, qrows)
            qsl = pl.ds(q * qrows, qrows)
            for d in (0, 1):
                comm = comm_r if d == 0 else comm_l
                col = col_r(2) if d == 0 else col_l(2)
                rdmas[(2, q, d)].wait_recv()
                out_vmem[d, q] = comm[3, s, rq] + x_vmem[qsl, pl.ds(col, half)]
                cp = pltpu.make_async_copy(
                    out_vmem.at[d, q],
                    out_hbm.at[qsl, pl.ds(d * half, half)],
                    out_sems.at[d, q],
                )
                cp.start()
                out_copies.append(cp)

        for r in rdmas.values():
            r.wait_send()
        for cp in out_copies:
            cp.wait()

    return pl.pallas_call(
        body,
        out_shape=jax.ShapeDtypeStruct((m, chunk), jnp.float32),
        in_specs=[pl.BlockSpec(memory_space=pl.ANY)],
        out_specs=pl.BlockSpec(memory_space=pl.ANY),
        scratch_shapes=[
            pltpu.VMEM((m, n), jnp.float32),
            pltpu.SemaphoreType.DMA,
            pltpu.VMEM((N_DEV, S, rows, half), jnp.float32),
            pltpu.VMEM((N_DEV, S, rows, half), jnp.float32),
            pltpu.SemaphoreType.DMA((2, S)),
            pltpu.SemaphoreType.DMA((2, S)),
            pltpu.SemaphoreType.DMA((2, S)),
            pltpu.SemaphoreType.DMA((2, S)),
            pltpu.SemaphoreType.DMA((Q,)),
            pltpu.SemaphoreType.DMA((Q,)),
            pltpu.SemaphoreType.DMA((Q,)),
            pltpu.SemaphoreType.DMA((Q,)),
            pltpu.VMEM((2, Q, qrows, half), jnp.float32),
            pltpu.SemaphoreType.DMA((2, Q)),
        ],
        compiler_params=pltpu.CompilerParams(collective_id=0),
    )(xs)
